# Initial kernel scaffold; baseline (speedup 1.0000x reference)
import jax
import jax.numpy as jnp
from jax.experimental import pallas as pl

_BM = 1152


def _vq_body(z_ref, cb_ref, emb_ref, q_ref):
    zb = z_ref[...]
    cb = cb_ref[...]
    zn = zb / jnp.maximum(
        jnp.sqrt(jnp.sum(zb * zb, axis=-1, keepdims=True)), 1e-12)
    en = cb / jnp.maximum(
        jnp.sqrt(jnp.sum(cb * cb, axis=-1, keepdims=True)), 1e-12)
    sims = jax.lax.dot_general(
        zn, en, (((1,), (1,)), ((), ())),
        precision=jax.lax.Precision.HIGHEST)
    k = sims.shape[1]
    col = jax.lax.broadcasted_iota(jnp.int32, sims.shape, 1)
    sims = jnp.where(col == 0, -jnp.inf, sims)
    m = jnp.max(sims, axis=1, keepdims=True)
    qi = jnp.min(jnp.where(sims == m, col, k), axis=1, keepdims=True)
    q_ref[...] = qi
    onehot = (col == qi).astype(jnp.float32)
    emb_ref[...] = jax.lax.dot_general(
        onehot, cb, (((1,), (0,)), ((), ())),
        precision=jax.lax.Precision.HIGHEST)


def kernel(z, codebook):
    shape = z.shape
    d = shape[-1]
    m = z.size // d
    k = codebook.shape[0]
    zf = z.reshape(m, d)
    grid = m // _BM
    emb, q = pl.pallas_call(
        _vq_body,
        grid=(grid,),
        in_specs=[
            pl.BlockSpec((_BM, d), lambda i: (i, 0)),
            pl.BlockSpec((k, d), lambda i: (0, 0)),
        ],
        out_specs=[
            pl.BlockSpec((_BM, d), lambda i: (i, 0)),
            pl.BlockSpec((_BM, 1), lambda i: (i, 0)),
        ],
        out_shape=[
            jax.ShapeDtypeStruct((m, d), jnp.float32),
            jax.ShapeDtypeStruct((m, 1), jnp.int32),
        ],
    )(zf, codebook)
    return emb.reshape(shape), q.reshape(shape[:-1] + (1,))


# speedup vs baseline: 1.8104x; 1.8104x over previous
import jax
import jax.numpy as jnp
from jax.experimental import pallas as pl

_BM = 1152


def _vq_body(z_ref, cb_ref, emb_ref, q_ref):
    zb = z_ref[...]
    cb = cb_ref[...]
    zn = zb / jnp.maximum(
        jnp.sqrt(jnp.sum(zb * zb, axis=-1, keepdims=True)), 1e-12)
    en = cb / jnp.maximum(
        jnp.sqrt(jnp.sum(cb * cb, axis=-1, keepdims=True)), 1e-12)
    sims = jax.lax.dot_general(
        zn, en, (((1,), (1,)), ((), ())),
        precision=jax.lax.Precision.DEFAULT)
    k = sims.shape[1]
    col = jax.lax.broadcasted_iota(jnp.int32, sims.shape, 1)
    sims = jnp.where(col == 0, -jnp.inf, sims)
    m = jnp.max(sims, axis=1, keepdims=True)
    qi = jnp.min(jnp.where(sims == m, col, k), axis=1, keepdims=True)
    q_ref[...] = qi
    onehot = (col == qi).astype(jnp.float32)
    emb_ref[...] = jax.lax.dot_general(
        onehot, cb, (((1,), (0,)), ((), ())),
        precision=jax.lax.Precision.HIGHEST)


def kernel(z, codebook):
    shape = z.shape
    d = shape[-1]
    m = z.size // d
    k = codebook.shape[0]
    zf = z.reshape(m, d)
    grid = m // _BM
    emb, q = pl.pallas_call(
        _vq_body,
        grid=(grid,),
        in_specs=[
            pl.BlockSpec((_BM, d), lambda i: (i, 0)),
            pl.BlockSpec((k, d), lambda i: (0, 0)),
        ],
        out_specs=[
            pl.BlockSpec((_BM, d), lambda i: (i, 0)),
            pl.BlockSpec((_BM, 1), lambda i: (i, 0)),
        ],
        out_shape=[
            jax.ShapeDtypeStruct((m, d), jnp.float32),
            jax.ShapeDtypeStruct((m, 1), jnp.int32),
        ],
    )(zf, codebook)
    return emb.reshape(shape), q.reshape(shape[:-1] + (1,))


# speedup vs baseline: 2.6786x; 1.4796x over previous
import functools

import jax
import jax.numpy as jnp
from jax import lax
from jax.experimental import pallas as pl
from jax.experimental.pallas import tpu as pltpu
from jax.experimental.pallas import tpu_sc as plsc

_BM = 1152


def _vq_body(z_ref, cb_ref, q_ref):
    zb = z_ref[...]
    cb = cb_ref[...]
    zn = zb / jnp.maximum(
        jnp.sqrt(jnp.sum(zb * zb, axis=-1, keepdims=True)), 1e-12)
    en = cb / jnp.maximum(
        jnp.sqrt(jnp.sum(cb * cb, axis=-1, keepdims=True)), 1e-12)
    sims = jax.lax.dot_general(
        zn, en, (((1,), (1,)), ((), ())),
        precision=jax.lax.Precision.DEFAULT)
    k = sims.shape[1]
    col = jax.lax.broadcasted_iota(jnp.int32, sims.shape, 1)
    sims = jnp.where(col == 0, -jnp.inf, sims)
    m = jnp.max(sims, axis=1, keepdims=True)
    qi = jnp.min(jnp.where(sims == m, col, k), axis=1, keepdims=True)
    q_ref[...] = qi


def _argmax_tc(zf, codebook):
    m, d = zf.shape
    k = codebook.shape[0]
    return pl.pallas_call(
        _vq_body,
        grid=(m // _BM,),
        in_specs=[
            pl.BlockSpec((_BM, d), lambda i: (i, 0)),
            pl.BlockSpec((k, d), lambda i: (0, 0)),
        ],
        out_specs=pl.BlockSpec((_BM, 1), lambda i: (i, 0)),
        out_shape=jax.ShapeDtypeStruct((m, 1), jnp.int32),
    )(zf, codebook)


def _gather_sc(codebook, idx_flat):
    m = idx_flat.shape[0]
    d = codebook.shape[1]
    nw = 2 * 16
    bpw = m // nw
    chunk = 96
    nch = bpw // chunk
    mesh = plsc.VectorSubcoreMesh(core_axis_name="c", subcore_axis_name="s")

    @functools.partial(
        pl.kernel, mesh=mesh,
        compiler_params=pltpu.CompilerParams(use_tc_tiling_on_sc=False),
        out_type=jax.ShapeDtypeStruct((m, d), jnp.float32),
        scratch_types=[
            pltpu.VMEM((bpw,), jnp.int32),
            pltpu.VMEM((bpw, d), jnp.float32),
            pltpu.SemaphoreType.DMA,
        ],
    )
    def gather(cb_hbm, idx_hbm, out_hbm, idx_v, rows_v, sem):
        wid = lax.axis_index("s") * 2 + lax.axis_index("c")
        base = wid * bpw
        pltpu.sync_copy(idx_hbm.at[pl.ds(base, bpw)], idx_v)
        copies = [
            pltpu.async_copy(
                cb_hbm.at[idx_v.at[pl.ds(j * chunk, chunk)]],
                rows_v.at[pl.ds(j * chunk, chunk), :],
                sem,
            )
            for j in range(nch)
        ]
        for c in copies:
            c.wait()
        pltpu.sync_copy(rows_v, out_hbm.at[pl.ds(base, bpw), :])

    return gather(codebook, idx_flat)


def kernel(z, codebook):
    shape = z.shape
    d = shape[-1]
    m = z.size // d
    zf = z.reshape(m, d)
    q = _argmax_tc(zf, codebook)
    emb = _gather_sc(codebook, q.reshape(m))
    return emb.reshape(shape), q.reshape(shape[:-1] + (1,))
